# baseline (speedup 1.0000x reference)
import jax
import jax.numpy as jnp
from jax.experimental import pallas as pl
from jax.experimental.pallas import tpu as pltpu

_ROWS = 4096
_COLS = 8192
_BLOCK_ROWS = 480


def _mask_mul_body(pruned_ref, img_ref, out_ref):
    mask = jnp.float32(1) - pruned_ref[...].astype(jnp.float32)
    out_ref[...] = img_ref[...] * mask


def kernel(image, pruned):
    img2d = image.reshape(_ROWS, _COLS)
    pruned2d = pruned.view(jnp.int8).reshape(1, _COLS)
    out = pl.pallas_call(
        _mask_mul_body,
        grid=(pl.cdiv(_ROWS, _BLOCK_ROWS),),
        in_specs=[
            pl.BlockSpec((1, _COLS), lambda i: (0, 0)),
            pl.BlockSpec((_BLOCK_ROWS, _COLS), lambda i: ((i - 1) % 9, 0)),
        ],
        out_specs=pl.BlockSpec((_BLOCK_ROWS, _COLS), lambda i: ((i - 1) % 9, 0)),
        out_shape=jax.ShapeDtypeStruct((_ROWS, _COLS), jnp.float32),
        compiler_params=pltpu.CompilerParams(
            vmem_limit_bytes=100 * 1024 * 1024,
        ),
    )(pruned2d, img2d)
    return out.reshape(image.shape)


# speedup vs baseline: 1.0160x; 1.0160x over previous
import jax
import jax.numpy as jnp
from jax.experimental import pallas as pl
from jax.experimental.pallas import tpu as pltpu

_ROWS = 4096
_COLS = 8192
_BLOCK_ROWS = 480


def _mask_mul_body(pruned_ref, img_ref, out_ref):
    mask = jnp.float32(1) - pruned_ref[...].astype(jnp.float32)
    out_ref[...] = img_ref[...] * mask


def kernel(image, pruned):
    img2d = image.reshape(_ROWS, _COLS)
    pruned2d = pruned.view(jnp.int8).reshape(1, _COLS)
    out = pl.pallas_call(
        _mask_mul_body,
        grid=(pl.cdiv(_ROWS, _BLOCK_ROWS),),
        in_specs=[
            pl.BlockSpec((1, _COLS), lambda i: (0, 0)),
            pl.BlockSpec((_BLOCK_ROWS, _COLS), lambda i: (i, 0)),
        ],
        out_specs=pl.BlockSpec((_BLOCK_ROWS, _COLS), lambda i: (i, 0)),
        out_shape=jax.ShapeDtypeStruct((_ROWS, _COLS), jnp.float32),
        compiler_params=pltpu.CompilerParams(
            vmem_limit_bytes=100 * 1024 * 1024,
        ),
    )(pruned2d, img2d)
    return out.reshape(image.shape)
